# Initial kernel scaffold; baseline (speedup 1.0000x reference)
import jax
import jax.numpy as jnp
from jax.experimental import pallas as pl


def kernel(d, c, confs, gt_confs, k):
    raise NotImplementedError("write your pallas kernel here")


# speedup vs baseline: 3.5017x; 3.5017x over previous
import jax
import jax.numpy as jnp
from jax.experimental import pallas as pl

N = 4096
DIM = 64
R = 32
B = 256
NB = N // B


def _stats_body(d_ref, c_ref, cf_ref, gf_ref, km_ref,
                r1_ref, ek_ref, em_ref, rc_ref, rg_ref, rd_ref):
    i = pl.program_id(0)
    dall = d_ref[...]
    dloc = d_ref[pl.ds(i * B, B), :]
    sq_all = jnp.sum(dall * dall, axis=1)
    sq_loc = jnp.sum(dloc * dloc, axis=1)
    prod = jax.lax.dot_general(dloc, dall, (((1,), (1,)), ((), ())),
                               preferred_element_type=jnp.float32)
    dist = sq_loc[:, None] + sq_all[None, :] - 2.0 * prod

    call = c_ref[0, :]
    cloc = c_ref[0, pl.ds(i * B, B)]
    eq = (cloc[:, None] == call[None, :]).astype(jnp.float32)
    col = jax.lax.broadcasted_iota(jnp.int32, (B, N), 1)
    km = km_ref[0, :]
    t_iota = jax.lax.broadcasted_iota(jnp.int32, (1, R), 1)

    def body(t, carry):
        dist, cum, mapacc, recacc, r1 = carry
        m = jnp.min(dist, axis=1, keepdims=True)
        ismin = dist == m
        idx = jnp.min(jnp.where(ismin, col, N), axis=1, keepdims=True)
        onehot = col == idx
        hit = jnp.sum(jnp.where(onehot, eq, 0.0), axis=1)
        dist = jnp.where(onehot, jnp.float32(jnp.inf), dist)
        w = jnp.where(t > 0, jnp.float32(1.0), jnp.float32(0.0))
        cum = cum + hit * w
        tf = jnp.maximum(t, 1).astype(jnp.float32)
        prec = cum / tf
        mapacc = mapacc + prec * hit * w
        kw = jnp.sum(jnp.where(t_iota == (t - 1), km[None, :], 0.0))
        recacc = recacc + hit * w * kw
        r1 = r1 + hit * jnp.where(t == 1, jnp.float32(1.0), jnp.float32(0.0))
        return dist, cum, mapacc, recacc, r1

    z = jnp.zeros((B,), jnp.float32)
    dist, cum, mapacc, recacc, r1 = jax.lax.fori_loop(
        0, R + 1, body, (dist, z, z, z, z))

    em = 1.0 - mapacc / jnp.float32(R)
    ek = 1.0 - (recacc > 0).astype(jnp.float32)

    gidx = i * B + jax.lax.broadcasted_iota(jnp.int32, (B, 1), 0)
    jlt = (col < gidx).astype(jnp.float32)

    def ranks(full_ref):
        a = full_ref[0, :][None, :]
        b = full_ref[0, pl.ds(i * B, B)][:, None]
        eqm = (a == b).astype(jnp.float32) * jlt
        lt = jnp.sum((a < b).astype(jnp.float32) + eqm, axis=1)
        gt = jnp.sum((a > b).astype(jnp.float32) + eqm, axis=1)
        return lt, gt

    rc_lt, rc_gt = ranks(cf_ref)
    rg_lt, _ = ranks(gf_ref)

    r1_ref[0, 0, :] = r1
    ek_ref[0, 0, :] = ek
    em_ref[0, 0, :] = em
    rc_ref[0, 0, :] = rc_lt
    rg_ref[0, 0, :] = rg_lt
    rd_ref[0, 0, :] = rc_gt


def _curves_body(r1_ref, ek_ref, em_ref, rc_ref, rg_ref, rd_ref,
                 o1_ref, ok_ref, om_ref, oc_ref):
    p = pl.program_id(0)
    pos = (p * B + jax.lax.broadcasted_iota(jnp.int32, (B, 1), 0)
           ).astype(jnp.float32)
    rd = rd_ref[0, :][None, :]
    mask = (rd <= pos).astype(jnp.float32)
    denom = pos[:, 0] + 1.0
    o1_ref[0, :] = jnp.sum(mask * r1_ref[0, :][None, :], axis=1) / denom
    ok_ref[0, :] = jnp.sum(mask * ek_ref[0, :][None, :], axis=1) / denom
    om_ref[0, :] = jnp.sum(mask * em_ref[0, :][None, :], axis=1) / denom

    @pl.when(p == 0)
    def _():
        rx = rc_ref[0, :]
        ry = rg_ref[0, :]
        rx = rx - jnp.mean(rx)
        ry = ry - jnp.mean(ry)
        val = (jnp.sum(rx * ry) /
               jnp.sqrt(jnp.sum(rx * rx) * jnp.sum(ry * ry)))
        oc_ref[...] = val.reshape(1, 1)


def kernel(d, c, confs, gt_confs, k):
    km = (jnp.arange(R) < k).astype(jnp.float32).reshape(1, R)
    c2 = c.reshape(1, N)
    cf = confs.reshape(1, N)
    gf = gt_confs.reshape(1, N)

    stat_shape = jax.ShapeDtypeStruct((NB, 1, B), jnp.float32)
    stat_spec = pl.BlockSpec((1, 1, B), lambda i: (i, 0, 0))
    full2 = pl.BlockSpec((1, N), lambda i: (0, 0))
    r1v, ekv, emv, rcv, rgv, rdv = pl.pallas_call(
        _stats_body,
        grid=(NB,),
        in_specs=[pl.BlockSpec((N, DIM), lambda i: (0, 0)),
                  full2, full2, full2,
                  pl.BlockSpec((1, R), lambda i: (0, 0))],
        out_specs=[stat_spec] * 6,
        out_shape=[stat_shape] * 6,
    )(d, c2, cf, gf, km)

    flats = [a.reshape(1, N) for a in (r1v, ekv, emv, rcv, rgv, rdv)]
    curve_shape = jax.ShapeDtypeStruct((1, N), jnp.float32)
    curve_spec = pl.BlockSpec((1, B), lambda p: (0, p))
    o1, ok, om, oc = pl.pallas_call(
        _curves_body,
        grid=(NB,),
        in_specs=[full2] * 6,
        out_specs=[curve_spec, curve_spec, curve_spec,
                   pl.BlockSpec((1, 1), lambda p: (0, 0))],
        out_shape=[curve_shape, curve_shape, curve_shape,
                   jax.ShapeDtypeStruct((1, 1), jnp.float32)],
    )(*flats)

    return (o1.reshape(N), oc.reshape(()), ok.reshape(N), om.reshape(N))


# speedup vs baseline: 3.8375x; 1.0959x over previous
import jax
import jax.numpy as jnp
from jax.experimental import pallas as pl

N = 4096
DIM = 64
R = 32
B = 256
NB = N // B


def _stats_body(d_ref, dp_ref, c_ref, cp_ref, cf_ref, gf_ref, km_ref,
                r1_ref, ek_ref, em_ref, rc_ref, rg_ref, rd_ref):
    i = pl.program_id(0)
    H = N // 2
    dp = dp_ref[...]
    dloc = d_ref[pl.ds(i * B, B), :]
    sq_p = jnp.sum(dp * dp, axis=1)
    sq_loc = jnp.sum(dloc * dloc, axis=1)
    prod = jax.lax.dot_general(dloc, dp, (((1,), (1,)), ((), ())),
                               preferred_element_type=jnp.float32)
    dist = sq_loc[:, None] + sq_p[None, :] - 2.0 * prod

    cp = cp_ref[0, :]
    cloc = c_ref[0, pl.ds(i * B, B)]
    eqa = (cloc[:, None] == cp[None, :H]).astype(jnp.float32)
    eqb = (cloc[:, None] == cp[None, H:]).astype(jnp.float32)
    a = dist[:, :H]
    b = dist[:, H:]
    le = a <= b
    lo = jnp.minimum(a, b)
    hi = jnp.maximum(a, b)
    eqlo = jnp.where(le, eqa, eqb)
    eqhi = jnp.where(le, eqb, eqa)

    slot = jax.lax.broadcasted_iota(jnp.int32, (B, H), 1)
    km = km_ref[0, :]
    t_iota = jax.lax.broadcasted_iota(jnp.int32, (1, R), 1)
    col = jax.lax.broadcasted_iota(jnp.int32, (B, N), 1)

    def body(t, carry):
        work, used, cum, mapacc, recacc, r1 = carry
        m = jnp.min(work, axis=1, keepdims=True)
        ismin = work == m
        sidx = jnp.min(jnp.where(ismin, slot, H), axis=1, keepdims=True)
        oh = slot == sidx
        usedb = used > 0.5
        hit = jnp.sum(jnp.where(oh, jnp.where(usedb, eqhi, eqlo), 0.0), axis=1)
        work = jnp.where(oh, jnp.where(usedb, jnp.float32(jnp.inf), hi), work)
        used = jnp.maximum(used, oh.astype(jnp.float32))
        w = jnp.where(t > 0, jnp.float32(1.0), jnp.float32(0.0))
        cum = cum + hit * w
        tf = jnp.maximum(t, 1).astype(jnp.float32)
        prec = cum / tf
        mapacc = mapacc + prec * hit * w
        kw = jnp.sum(jnp.where(t_iota == (t - 1), km[None, :], 0.0))
        recacc = recacc + hit * w * kw
        r1 = r1 + hit * jnp.where(t == 1, jnp.float32(1.0), jnp.float32(0.0))
        return work, used, cum, mapacc, recacc, r1

    z = jnp.zeros((B,), jnp.float32)
    nouse = jnp.zeros((B, H), jnp.float32)
    _, _, cum, mapacc, recacc, r1 = jax.lax.fori_loop(
        0, R + 1, body, (lo, nouse, z, z, z, z))

    em = 1.0 - mapacc / jnp.float32(R)
    ek = 1.0 - (recacc > 0).astype(jnp.float32)

    gidx = i * B + jax.lax.broadcasted_iota(jnp.int32, (B, 1), 0)
    jlt = (col < gidx).astype(jnp.float32)

    def ranks(full_ref):
        a = full_ref[0, :][None, :]
        b = full_ref[0, pl.ds(i * B, B)][:, None]
        eqm = (a == b).astype(jnp.float32) * jlt
        lt = jnp.sum((a < b).astype(jnp.float32) + eqm, axis=1)
        gt = jnp.sum((a > b).astype(jnp.float32) + eqm, axis=1)
        return lt, gt

    rc_lt, rc_gt = ranks(cf_ref)
    rg_lt, _ = ranks(gf_ref)

    r1_ref[0, 0, :] = r1
    ek_ref[0, 0, :] = ek
    em_ref[0, 0, :] = em
    rc_ref[0, 0, :] = rc_lt
    rg_ref[0, 0, :] = rg_lt
    rd_ref[0, 0, :] = rc_gt


def _curves_body(r1_ref, ek_ref, em_ref, rc_ref, rg_ref, rd_ref,
                 o1_ref, ok_ref, om_ref, oc_ref):
    p = pl.program_id(0)
    pos = (p * B + jax.lax.broadcasted_iota(jnp.int32, (B, 1), 0)
           ).astype(jnp.float32)
    rd = rd_ref[0, :][None, :]
    mask = (rd <= pos).astype(jnp.float32)
    denom = pos[:, 0] + 1.0
    o1_ref[0, :] = jnp.sum(mask * r1_ref[0, :][None, :], axis=1) / denom
    ok_ref[0, :] = jnp.sum(mask * ek_ref[0, :][None, :], axis=1) / denom
    om_ref[0, :] = jnp.sum(mask * em_ref[0, :][None, :], axis=1) / denom

    @pl.when(p == 0)
    def _():
        rx = rc_ref[0, :]
        ry = rg_ref[0, :]
        rx = rx - jnp.mean(rx)
        ry = ry - jnp.mean(ry)
        val = (jnp.sum(rx * ry) /
               jnp.sqrt(jnp.sum(rx * rx) * jnp.sum(ry * ry)))
        oc_ref[...] = val.reshape(1, 1)


def kernel(d, c, confs, gt_confs, k):
    km = (jnp.arange(R) < k).astype(jnp.float32).reshape(1, R)
    perm = jnp.concatenate([jnp.arange(0, N, 2), jnp.arange(1, N, 2)])
    dp = d[perm]
    cp = c[perm].reshape(1, N)
    c2 = c.reshape(1, N)
    cf = confs.reshape(1, N)
    gf = gt_confs.reshape(1, N)

    stat_shape = jax.ShapeDtypeStruct((NB, 1, B), jnp.float32)
    stat_spec = pl.BlockSpec((1, 1, B), lambda i: (i, 0, 0))
    full2 = pl.BlockSpec((1, N), lambda i: (0, 0))
    r1v, ekv, emv, rcv, rgv, rdv = pl.pallas_call(
        _stats_body,
        grid=(NB,),
        in_specs=[pl.BlockSpec((N, DIM), lambda i: (0, 0)),
                  pl.BlockSpec((N, DIM), lambda i: (0, 0)),
                  full2, full2, full2, full2,
                  pl.BlockSpec((1, R), lambda i: (0, 0))],
        out_specs=[stat_spec] * 6,
        out_shape=[stat_shape] * 6,
    )(d, dp, c2, cp, cf, gf, km)

    flats = [a.reshape(1, N) for a in (r1v, ekv, emv, rcv, rgv, rdv)]
    curve_shape = jax.ShapeDtypeStruct((1, N), jnp.float32)
    curve_spec = pl.BlockSpec((1, B), lambda p: (0, p))
    o1, ok, om, oc = pl.pallas_call(
        _curves_body,
        grid=(NB,),
        in_specs=[full2] * 6,
        out_specs=[curve_spec, curve_spec, curve_spec,
                   pl.BlockSpec((1, 1), lambda p: (0, 0))],
        out_shape=[curve_shape, curve_shape, curve_shape,
                   jax.ShapeDtypeStruct((1, 1), jnp.float32)],
    )(*flats)

    return (o1.reshape(N), oc.reshape(()), ok.reshape(N), om.reshape(N))


# speedup vs baseline: 3.9161x; 1.0205x over previous
import jax
import jax.numpy as jnp
from jax.experimental import pallas as pl

N = 4096
DIM = 64
R = 32
B = 256
NB = N // B


def _stats_body(d_ref, dp_ref, c_ref, cp_ref, cf_ref, gf_ref, km_ref,
                r1_ref, ek_ref, em_ref, rc_ref, rg_ref, rd_ref):
    i = pl.program_id(0)
    H = N // 2
    dp = dp_ref[...]
    dloc = d_ref[pl.ds(i * B, B), :]
    sq_p = jnp.sum(dp * dp, axis=1)
    sq_loc = jnp.sum(dloc * dloc, axis=1)
    prod = jax.lax.dot_general(dloc, dp, (((1,), (1,)), ((), ())),
                               preferred_element_type=jnp.float32)
    dist = sq_loc[:, None] + sq_p[None, :] - 2.0 * prod

    cp = cp_ref[0, :]
    cloc = c_ref[0, pl.ds(i * B, B)]
    eqa = (cloc[:, None] == cp[None, :H]).astype(jnp.int32)
    eqb = (cloc[:, None] == cp[None, H:]).astype(jnp.int32)
    a = dist[:, :H]
    b = dist[:, H:]
    le = a <= b
    lo = jnp.minimum(a, b)
    hi = jnp.maximum(a, b)
    slot2 = 2 * jax.lax.broadcasted_iota(jnp.int32, (B, H), 1)
    klo = slot2 + jnp.where(le, eqa, eqb)
    khi = slot2 + jnp.where(le, eqb, eqa)
    BIGK = jnp.int32(2 * H + 2)

    km = km_ref[0, :]
    t_iota = jax.lax.broadcasted_iota(jnp.int32, (1, R), 1)
    col = jax.lax.broadcasted_iota(jnp.int32, (B, N), 1)

    def body(t, carry):
        work, nxt, kk, cum, mapacc, recacc, r1 = carry
        m = jnp.min(work, axis=1, keepdims=True)
        skey = jnp.min(jnp.where(work == m, kk, BIGK), axis=1, keepdims=True)
        oh = kk == skey
        hit = (skey & 1).astype(jnp.float32)[:, 0]
        work = jnp.where(oh, nxt, work)
        nxt = jnp.where(oh, jnp.float32(jnp.inf), nxt)
        kk = jnp.where(oh, khi, kk)
        w = jnp.where(t > 0, jnp.float32(1.0), jnp.float32(0.0))
        cum = cum + hit * w
        tf = jnp.maximum(t, 1).astype(jnp.float32)
        prec = cum / tf
        mapacc = mapacc + prec * hit * w
        kw = jnp.sum(jnp.where(t_iota == (t - 1), km[None, :], 0.0))
        recacc = recacc + hit * w * kw
        r1 = r1 + hit * jnp.where(t == 1, jnp.float32(1.0), jnp.float32(0.0))
        return work, nxt, kk, cum, mapacc, recacc, r1

    z = jnp.zeros((B,), jnp.float32)
    _, _, _, cum, mapacc, recacc, r1 = jax.lax.fori_loop(
        0, R + 1, body, (lo, hi, klo, z, z, z, z))

    em = 1.0 - mapacc / jnp.float32(R)
    ek = 1.0 - (recacc > 0).astype(jnp.float32)

    gidx = i * B + jax.lax.broadcasted_iota(jnp.int32, (B, 1), 0)
    jlt = (col < gidx).astype(jnp.float32)

    def ranks(full_ref):
        a = full_ref[0, :][None, :]
        b = full_ref[0, pl.ds(i * B, B)][:, None]
        eqm = (a == b).astype(jnp.float32) * jlt
        lt = jnp.sum((a < b).astype(jnp.float32) + eqm, axis=1)
        gt = jnp.sum((a > b).astype(jnp.float32) + eqm, axis=1)
        return lt, gt

    rc_lt, rc_gt = ranks(cf_ref)
    rg_lt, _ = ranks(gf_ref)

    r1_ref[0, 0, :] = r1
    ek_ref[0, 0, :] = ek
    em_ref[0, 0, :] = em
    rc_ref[0, 0, :] = rc_lt
    rg_ref[0, 0, :] = rg_lt
    rd_ref[0, 0, :] = rc_gt


def _curves_body(r1_ref, ek_ref, em_ref, rc_ref, rg_ref, rd_ref,
                 o1_ref, ok_ref, om_ref, oc_ref):
    p = pl.program_id(0)
    pos = (p * B + jax.lax.broadcasted_iota(jnp.int32, (B, 1), 0)
           ).astype(jnp.float32)
    rd = rd_ref[0, :][None, :]
    mask = (rd <= pos).astype(jnp.float32)
    denom = pos[:, 0] + 1.0
    o1_ref[0, :] = jnp.sum(mask * r1_ref[0, :][None, :], axis=1) / denom
    ok_ref[0, :] = jnp.sum(mask * ek_ref[0, :][None, :], axis=1) / denom
    om_ref[0, :] = jnp.sum(mask * em_ref[0, :][None, :], axis=1) / denom

    @pl.when(p == 0)
    def _():
        rx = rc_ref[0, :]
        ry = rg_ref[0, :]
        rx = rx - jnp.mean(rx)
        ry = ry - jnp.mean(ry)
        val = (jnp.sum(rx * ry) /
               jnp.sqrt(jnp.sum(rx * rx) * jnp.sum(ry * ry)))
        oc_ref[...] = val.reshape(1, 1)


def kernel(d, c, confs, gt_confs, k):
    km = (jnp.arange(R) < k).astype(jnp.float32).reshape(1, R)
    perm = jnp.concatenate([jnp.arange(0, N, 2), jnp.arange(1, N, 2)])
    dp = d[perm]
    cp = c[perm].reshape(1, N)
    c2 = c.reshape(1, N)
    cf = confs.reshape(1, N)
    gf = gt_confs.reshape(1, N)

    stat_shape = jax.ShapeDtypeStruct((NB, 1, B), jnp.float32)
    stat_spec = pl.BlockSpec((1, 1, B), lambda i: (i, 0, 0))
    full2 = pl.BlockSpec((1, N), lambda i: (0, 0))
    r1v, ekv, emv, rcv, rgv, rdv = pl.pallas_call(
        _stats_body,
        grid=(NB,),
        in_specs=[pl.BlockSpec((N, DIM), lambda i: (0, 0)),
                  pl.BlockSpec((N, DIM), lambda i: (0, 0)),
                  full2, full2, full2, full2,
                  pl.BlockSpec((1, R), lambda i: (0, 0))],
        out_specs=[stat_spec] * 6,
        out_shape=[stat_shape] * 6,
    )(d, dp, c2, cp, cf, gf, km)

    flats = [a.reshape(1, N) for a in (r1v, ekv, emv, rcv, rgv, rdv)]
    curve_shape = jax.ShapeDtypeStruct((1, N), jnp.float32)
    curve_spec = pl.BlockSpec((1, B), lambda p: (0, p))
    o1, ok, om, oc = pl.pallas_call(
        _curves_body,
        grid=(NB,),
        in_specs=[full2] * 6,
        out_specs=[curve_spec, curve_spec, curve_spec,
                   pl.BlockSpec((1, 1), lambda p: (0, 0))],
        out_shape=[curve_shape, curve_shape, curve_shape,
                   jax.ShapeDtypeStruct((1, 1), jnp.float32)],
    )(*flats)

    return (o1.reshape(N), oc.reshape(()), ok.reshape(N), om.reshape(N))


# speedup vs baseline: 4.0033x; 1.0223x over previous
import jax
import jax.numpy as jnp
from jax.experimental import pallas as pl

N = 4096
DIM = 64
R = 32
B = 256
NB = N // B


def _stats_body(d_ref, c_ref, cf_ref, gf_ref, km_ref,
                r1_ref, ek_ref, em_ref, rc_ref, rg_ref, rd_ref):
    i = pl.program_id(0)
    H = N // 2
    dall = d_ref[...]
    dloc = d_ref[pl.ds(i * B, B), :]
    sq_all = jnp.sum(dall * dall, axis=1)
    sq_loc = jnp.sum(dloc * dloc, axis=1)
    prod = jax.lax.dot_general(dloc, dall, (((1,), (1,)), ((), ())),
                               preferred_element_type=jnp.float32)
    dist = sq_loc[:, None] + sq_all[None, :] - 2.0 * prod

    call = c_ref[0, :]
    cloc = c_ref[0, pl.ds(i * B, B)]
    eqa = (cloc[:, None] == call[None, :H]).astype(jnp.int32)
    eqb = (cloc[:, None] == call[None, H:]).astype(jnp.int32)
    a = dist[:, :H]
    b = dist[:, H:]
    le = a <= b
    lo = jnp.minimum(a, b)
    hi = jnp.maximum(a, b)
    slot2 = 2 * jax.lax.broadcasted_iota(jnp.int32, (B, H), 1)
    klo = slot2 + jnp.where(le, eqa, eqb + 2 * H)
    khi = slot2 + jnp.where(le, eqb + 2 * H, eqa)
    BIGK = jnp.int32(4 * H + 2)

    km = km_ref[0, :]
    t_iota = jax.lax.broadcasted_iota(jnp.int32, (1, R), 1)
    col = jax.lax.broadcasted_iota(jnp.int32, (B, N), 1)

    def body(t, carry):
        work, nxt, kk, cum, mapacc, recacc, r1 = carry
        m = jnp.min(work, axis=1, keepdims=True)
        skey = jnp.min(jnp.where(work == m, kk, BIGK), axis=1, keepdims=True)
        oh = kk == skey
        hit = (skey & 1).astype(jnp.float32)[:, 0]
        work = jnp.where(oh, nxt, work)
        nxt = jnp.where(oh, jnp.float32(jnp.inf), nxt)
        kk = jnp.where(oh, khi, kk)
        w = jnp.where(t > 0, jnp.float32(1.0), jnp.float32(0.0))
        cum = cum + hit * w
        tf = jnp.maximum(t, 1).astype(jnp.float32)
        prec = cum / tf
        mapacc = mapacc + prec * hit * w
        kw = jnp.sum(jnp.where(t_iota == (t - 1), km[None, :], 0.0))
        recacc = recacc + hit * w * kw
        r1 = r1 + hit * jnp.where(t == 1, jnp.float32(1.0), jnp.float32(0.0))
        return work, nxt, kk, cum, mapacc, recacc, r1

    z = jnp.zeros((B,), jnp.float32)
    _, _, _, cum, mapacc, recacc, r1 = jax.lax.fori_loop(
        0, R + 1, body, (lo, hi, klo, z, z, z, z))

    em = 1.0 - mapacc / jnp.float32(R)
    ek = 1.0 - (recacc > 0).astype(jnp.float32)

    gidx = i * B + jax.lax.broadcasted_iota(jnp.int32, (B, 1), 0)
    jlt = (col < gidx).astype(jnp.float32)

    def ranks(full_ref):
        a = full_ref[0, :][None, :]
        b = full_ref[0, pl.ds(i * B, B)][:, None]
        eqm = (a == b).astype(jnp.float32) * jlt
        lt = jnp.sum((a < b).astype(jnp.float32) + eqm, axis=1)
        gt = jnp.sum((a > b).astype(jnp.float32) + eqm, axis=1)
        return lt, gt

    rc_lt, rc_gt = ranks(cf_ref)
    rg_lt, _ = ranks(gf_ref)

    r1_ref[0, 0, :] = r1
    ek_ref[0, 0, :] = ek
    em_ref[0, 0, :] = em
    rc_ref[0, 0, :] = rc_lt
    rg_ref[0, 0, :] = rg_lt
    rd_ref[0, 0, :] = rc_gt


def _curves_body(r1_ref, ek_ref, em_ref, rc_ref, rg_ref, rd_ref,
                 o1_ref, ok_ref, om_ref, oc_ref):
    p = pl.program_id(0)
    pos = (p * B + jax.lax.broadcasted_iota(jnp.int32, (B, 1), 0)
           ).astype(jnp.float32)
    rd = rd_ref[0, :][None, :]
    mask = (rd <= pos).astype(jnp.float32)
    denom = pos[:, 0] + 1.0
    o1_ref[0, :] = jnp.sum(mask * r1_ref[0, :][None, :], axis=1) / denom
    ok_ref[0, :] = jnp.sum(mask * ek_ref[0, :][None, :], axis=1) / denom
    om_ref[0, :] = jnp.sum(mask * em_ref[0, :][None, :], axis=1) / denom

    @pl.when(p == 0)
    def _():
        rx = rc_ref[0, :]
        ry = rg_ref[0, :]
        rx = rx - jnp.mean(rx)
        ry = ry - jnp.mean(ry)
        val = (jnp.sum(rx * ry) /
               jnp.sqrt(jnp.sum(rx * rx) * jnp.sum(ry * ry)))
        oc_ref[...] = val.reshape(1, 1)


def kernel(d, c, confs, gt_confs, k):
    km = (jnp.arange(R) < k).astype(jnp.float32).reshape(1, R)
    c2 = c.reshape(1, N)
    cf = confs.reshape(1, N)
    gf = gt_confs.reshape(1, N)

    stat_shape = jax.ShapeDtypeStruct((NB, 1, B), jnp.float32)
    stat_spec = pl.BlockSpec((1, 1, B), lambda i: (i, 0, 0))
    full2 = pl.BlockSpec((1, N), lambda i: (0, 0))
    r1v, ekv, emv, rcv, rgv, rdv = pl.pallas_call(
        _stats_body,
        grid=(NB,),
        in_specs=[pl.BlockSpec((N, DIM), lambda i: (0, 0)),
                  full2, full2, full2,
                  pl.BlockSpec((1, R), lambda i: (0, 0))],
        out_specs=[stat_spec] * 6,
        out_shape=[stat_shape] * 6,
    )(d, c2, cf, gf, km)

    flats = [a.reshape(1, N) for a in (r1v, ekv, emv, rcv, rgv, rdv)]
    curve_shape = jax.ShapeDtypeStruct((1, N), jnp.float32)
    curve_spec = pl.BlockSpec((1, B), lambda p: (0, p))
    o1, ok, om, oc = pl.pallas_call(
        _curves_body,
        grid=(NB,),
        in_specs=[full2] * 6,
        out_specs=[curve_spec, curve_spec, curve_spec,
                   pl.BlockSpec((1, 1), lambda p: (0, 0))],
        out_shape=[curve_shape, curve_shape, curve_shape,
                   jax.ShapeDtypeStruct((1, 1), jnp.float32)],
    )(*flats)

    return (o1.reshape(N), oc.reshape(()), ok.reshape(N), om.reshape(N))


# speedup vs baseline: 4.1855x; 1.0455x over previous
import jax
import jax.numpy as jnp
from jax.experimental import pallas as pl

N = 4096
DIM = 64
R = 32
B = 256
NB = N // B


def _stats_body(d_ref, c_ref, cf_ref, gf_ref, km_ref,
                r1_ref, ek_ref, em_ref, rc_ref, rg_ref, rd_ref):
    i = pl.program_id(0)
    H = N // 2
    dall = d_ref[...]
    dloc = d_ref[pl.ds(i * B, B), :]
    sq_all = jnp.sum(dall * dall, axis=1)
    sq_loc = jnp.sum(dloc * dloc, axis=1)
    prod = jax.lax.dot_general(dloc, dall, (((1,), (1,)), ((), ())),
                               preferred_element_type=jnp.float32)
    dist = sq_loc[:, None] + sq_all[None, :] - 2.0 * prod

    call = c_ref[0, :]
    cloc = c_ref[0, pl.ds(i * B, B)]
    eqa = (cloc[:, None] == call[None, :H]).astype(jnp.int32)
    eqb = (cloc[:, None] == call[None, H:]).astype(jnp.int32)
    a = dist[:, :H]
    b = dist[:, H:]
    le = a <= b
    lo = jnp.minimum(a, b)
    hi = jnp.maximum(a, b)
    slot2 = 2 * jax.lax.broadcasted_iota(jnp.int32, (B, H), 1)
    klo = slot2 + jnp.where(le, eqa, eqb + 2 * H)
    khi = slot2 + jnp.where(le, eqb + 2 * H, eqa)
    BIGK = jnp.int32(4 * H + 2)

    km = km_ref[0, :]
    t_iota = jax.lax.broadcasted_iota(jnp.int32, (1, R), 1)
    col = jax.lax.broadcasted_iota(jnp.int32, (B, N), 1)

    def body(t, carry):
        work, kk, cum, mapacc, recacc, r1 = carry
        m = jnp.min(work, axis=1, keepdims=True)
        skey = jnp.min(jnp.where(work == m, kk, BIGK), axis=1, keepdims=True)
        oh = kk == skey
        hit = (skey & 1).astype(jnp.float32)[:, 0]
        work = jnp.where(oh, jnp.where(kk == khi, jnp.float32(jnp.inf), hi),
                         work)
        kk = jnp.where(oh, khi, kk)
        w = jnp.where(t > 0, jnp.float32(1.0), jnp.float32(0.0))
        cum = cum + hit * w
        tf = jnp.maximum(t, 1).astype(jnp.float32)
        prec = cum / tf
        mapacc = mapacc + prec * hit * w
        kw = jnp.sum(jnp.where(t_iota == (t - 1), km[None, :], 0.0))
        recacc = recacc + hit * w * kw
        r1 = r1 + hit * jnp.where(t == 1, jnp.float32(1.0), jnp.float32(0.0))
        return work, kk, cum, mapacc, recacc, r1

    z = jnp.zeros((B,), jnp.float32)
    _, _, cum, mapacc, recacc, r1 = jax.lax.fori_loop(
        0, R + 1, body, (lo, klo, z, z, z, z))

    em = 1.0 - mapacc / jnp.float32(R)
    ek = 1.0 - (recacc > 0).astype(jnp.float32)

    gidx = i * B + jax.lax.broadcasted_iota(jnp.int32, (B, 1), 0)
    jlt = (col < gidx).astype(jnp.float32)

    def ranks(full_ref):
        a = full_ref[0, :][None, :]
        b = full_ref[0, pl.ds(i * B, B)][:, None]
        eqm = (a == b).astype(jnp.float32) * jlt
        lt = jnp.sum((a < b).astype(jnp.float32) + eqm, axis=1)
        gt = jnp.sum((a > b).astype(jnp.float32) + eqm, axis=1)
        return lt, gt

    rc_lt, rc_gt = ranks(cf_ref)
    rg_lt, _ = ranks(gf_ref)

    r1_ref[0, 0, :] = r1
    ek_ref[0, 0, :] = ek
    em_ref[0, 0, :] = em
    rc_ref[0, 0, :] = rc_lt
    rg_ref[0, 0, :] = rg_lt
    rd_ref[0, 0, :] = rc_gt


def _curves_body(r1_ref, ek_ref, em_ref, rc_ref, rg_ref, rd_ref,
                 o1_ref, ok_ref, om_ref, oc_ref):
    p = pl.program_id(0)
    pos = (p * B + jax.lax.broadcasted_iota(jnp.int32, (B, 1), 0)
           ).astype(jnp.float32)
    rd = rd_ref[0, :][None, :]
    mask = (rd <= pos).astype(jnp.float32)
    denom = pos[:, 0] + 1.0
    o1_ref[0, :] = jnp.sum(mask * r1_ref[0, :][None, :], axis=1) / denom
    ok_ref[0, :] = jnp.sum(mask * ek_ref[0, :][None, :], axis=1) / denom
    om_ref[0, :] = jnp.sum(mask * em_ref[0, :][None, :], axis=1) / denom

    @pl.when(p == 0)
    def _():
        rx = rc_ref[0, :]
        ry = rg_ref[0, :]
        rx = rx - jnp.mean(rx)
        ry = ry - jnp.mean(ry)
        val = (jnp.sum(rx * ry) /
               jnp.sqrt(jnp.sum(rx * rx) * jnp.sum(ry * ry)))
        oc_ref[...] = val.reshape(1, 1)


def kernel(d, c, confs, gt_confs, k):
    km = (jnp.arange(R) < k).astype(jnp.float32).reshape(1, R)
    c2 = c.reshape(1, N)
    cf = confs.reshape(1, N)
    gf = gt_confs.reshape(1, N)

    stat_shape = jax.ShapeDtypeStruct((NB, 1, B), jnp.float32)
    stat_spec = pl.BlockSpec((1, 1, B), lambda i: (i, 0, 0))
    full2 = pl.BlockSpec((1, N), lambda i: (0, 0))
    r1v, ekv, emv, rcv, rgv, rdv = pl.pallas_call(
        _stats_body,
        grid=(NB,),
        in_specs=[pl.BlockSpec((N, DIM), lambda i: (0, 0)),
                  full2, full2, full2,
                  pl.BlockSpec((1, R), lambda i: (0, 0))],
        out_specs=[stat_spec] * 6,
        out_shape=[stat_shape] * 6,
    )(d, c2, cf, gf, km)

    flats = [a.reshape(1, N) for a in (r1v, ekv, emv, rcv, rgv, rdv)]
    curve_shape = jax.ShapeDtypeStruct((1, N), jnp.float32)
    curve_spec = pl.BlockSpec((1, B), lambda p: (0, p))
    o1, ok, om, oc = pl.pallas_call(
        _curves_body,
        grid=(NB,),
        in_specs=[full2] * 6,
        out_specs=[curve_spec, curve_spec, curve_spec,
                   pl.BlockSpec((1, 1), lambda p: (0, 0))],
        out_shape=[curve_shape, curve_shape, curve_shape,
                   jax.ShapeDtypeStruct((1, 1), jnp.float32)],
    )(*flats)

    return (o1.reshape(N), oc.reshape(()), ok.reshape(N), om.reshape(N))


# speedup vs baseline: 4.4471x; 1.0625x over previous
import jax
import jax.numpy as jnp
from jax.experimental import pallas as pl

N = 4096
DIM = 64
R = 32
B = 512
NB = N // B


def _stats_body(d_ref, c_ref, cf_ref, gf_ref, km_ref,
                r1_ref, ek_ref, em_ref, rc_ref, rg_ref, rd_ref):
    i = pl.program_id(0)
    H = N // 2
    dall = d_ref[...]
    dloc = d_ref[pl.ds(i * B, B), :]
    sq_all = jnp.sum(dall * dall, axis=1)
    sq_loc = jnp.sum(dloc * dloc, axis=1)
    prod = jax.lax.dot_general(dloc, dall, (((1,), (1,)), ((), ())),
                               preferred_element_type=jnp.float32)
    dist = sq_loc[:, None] + sq_all[None, :] - 2.0 * prod

    call = c_ref[0, :]
    cloc = c_ref[0, pl.ds(i * B, B)]
    eqa = (cloc[:, None] == call[None, :H]).astype(jnp.int32)
    eqb = (cloc[:, None] == call[None, H:]).astype(jnp.int32)
    a = dist[:, :H]
    b = dist[:, H:]
    le = a <= b
    lo = jnp.minimum(a, b)
    hi = jnp.maximum(a, b)
    slot2 = 2 * jax.lax.broadcasted_iota(jnp.int32, (B, H), 1)
    klo = slot2 + jnp.where(le, eqa, eqb + 2 * H)
    khi = slot2 + jnp.where(le, eqb + 2 * H, eqa)
    BIGK = jnp.int32(4 * H + 2)

    km = km_ref[0, :]
    t_iota = jax.lax.broadcasted_iota(jnp.int32, (1, R), 1)
    col = jax.lax.broadcasted_iota(jnp.int32, (B, N), 1)

    def body(t, carry):
        work, kk, cum, mapacc, recacc, r1 = carry
        m = jnp.min(work, axis=1, keepdims=True)
        skey = jnp.min(jnp.where(work == m, kk, BIGK), axis=1, keepdims=True)
        oh = kk == skey
        hit = (skey & 1).astype(jnp.float32)[:, 0]
        work = jnp.where(oh, jnp.where(kk == khi, jnp.float32(jnp.inf), hi),
                         work)
        kk = jnp.where(oh, khi, kk)
        w = jnp.where(t > 0, jnp.float32(1.0), jnp.float32(0.0))
        cum = cum + hit * w
        tf = jnp.maximum(t, 1).astype(jnp.float32)
        prec = cum / tf
        mapacc = mapacc + prec * hit * w
        kw = jnp.sum(jnp.where(t_iota == (t - 1), km[None, :], 0.0))
        recacc = recacc + hit * w * kw
        r1 = r1 + hit * jnp.where(t == 1, jnp.float32(1.0), jnp.float32(0.0))
        return work, kk, cum, mapacc, recacc, r1

    z = jnp.zeros((B,), jnp.float32)
    _, _, cum, mapacc, recacc, r1 = jax.lax.fori_loop(
        0, R + 1, body, (lo, klo, z, z, z, z))

    em = 1.0 - mapacc / jnp.float32(R)
    ek = 1.0 - (recacc > 0).astype(jnp.float32)

    gidx = i * B + jax.lax.broadcasted_iota(jnp.int32, (B, 1), 0)
    jlt = (col < gidx).astype(jnp.float32)

    def ranks(full_ref):
        a = full_ref[0, :][None, :]
        b = full_ref[0, pl.ds(i * B, B)][:, None]
        eqm = (a == b).astype(jnp.float32) * jlt
        lt = jnp.sum((a < b).astype(jnp.float32) + eqm, axis=1)
        gt = jnp.sum((a > b).astype(jnp.float32) + eqm, axis=1)
        return lt, gt

    rc_lt, rc_gt = ranks(cf_ref)
    rg_lt, _ = ranks(gf_ref)

    r1_ref[0, 0, :] = r1
    ek_ref[0, 0, :] = ek
    em_ref[0, 0, :] = em
    rc_ref[0, 0, :] = rc_lt
    rg_ref[0, 0, :] = rg_lt
    rd_ref[0, 0, :] = rc_gt


def _curves_body(r1_ref, ek_ref, em_ref, rc_ref, rg_ref, rd_ref,
                 o1_ref, ok_ref, om_ref, oc_ref):
    p = pl.program_id(0)
    pos = (p * B + jax.lax.broadcasted_iota(jnp.int32, (B, 1), 0)
           ).astype(jnp.float32)
    rd = rd_ref[0, :][None, :]
    mask = (rd <= pos).astype(jnp.float32)
    denom = pos[:, 0] + 1.0
    o1_ref[0, :] = jnp.sum(mask * r1_ref[0, :][None, :], axis=1) / denom
    ok_ref[0, :] = jnp.sum(mask * ek_ref[0, :][None, :], axis=1) / denom
    om_ref[0, :] = jnp.sum(mask * em_ref[0, :][None, :], axis=1) / denom

    @pl.when(p == 0)
    def _():
        rx = rc_ref[0, :]
        ry = rg_ref[0, :]
        rx = rx - jnp.mean(rx)
        ry = ry - jnp.mean(ry)
        val = (jnp.sum(rx * ry) /
               jnp.sqrt(jnp.sum(rx * rx) * jnp.sum(ry * ry)))
        oc_ref[...] = val.reshape(1, 1)


def kernel(d, c, confs, gt_confs, k):
    km = (jnp.arange(R) < k).astype(jnp.float32).reshape(1, R)
    c2 = c.reshape(1, N)
    cf = confs.reshape(1, N)
    gf = gt_confs.reshape(1, N)

    stat_shape = jax.ShapeDtypeStruct((NB, 1, B), jnp.float32)
    stat_spec = pl.BlockSpec((1, 1, B), lambda i: (i, 0, 0))
    full2 = pl.BlockSpec((1, N), lambda i: (0, 0))
    r1v, ekv, emv, rcv, rgv, rdv = pl.pallas_call(
        _stats_body,
        grid=(NB,),
        in_specs=[pl.BlockSpec((N, DIM), lambda i: (0, 0)),
                  full2, full2, full2,
                  pl.BlockSpec((1, R), lambda i: (0, 0))],
        out_specs=[stat_spec] * 6,
        out_shape=[stat_shape] * 6,
    )(d, c2, cf, gf, km)

    flats = [a.reshape(1, N) for a in (r1v, ekv, emv, rcv, rgv, rdv)]
    curve_shape = jax.ShapeDtypeStruct((1, N), jnp.float32)
    curve_spec = pl.BlockSpec((1, B), lambda p: (0, p))
    o1, ok, om, oc = pl.pallas_call(
        _curves_body,
        grid=(NB,),
        in_specs=[full2] * 6,
        out_specs=[curve_spec, curve_spec, curve_spec,
                   pl.BlockSpec((1, 1), lambda p: (0, 0))],
        out_shape=[curve_shape, curve_shape, curve_shape,
                   jax.ShapeDtypeStruct((1, 1), jnp.float32)],
    )(*flats)

    return (o1.reshape(N), oc.reshape(()), ok.reshape(N), om.reshape(N))
